# Initial kernel scaffold; baseline (speedup 1.0000x reference)
import jax
import jax.numpy as jnp
from jax.experimental import pallas as pl


def kernel(hidden_states, Wr, W1, b1, W2, b2):
    raise NotImplementedError("write your pallas kernel here")


# speedup vs baseline: 1.3150x; 1.3150x over previous
import functools

import jax
import jax.numpy as jnp
from jax import lax
from jax.experimental import pallas as pl
from jax.experimental.pallas import tpu as pltpu
from jax.experimental.pallas import tpu_sc as plsc

T = 2048
H = 2048
E = 8
K = 2
DFF = 8192

TT = 256
NT_R = T // TT

TM = 512
P = 4096 + E * TM // 1
NT = P // TM
FF = 1024
NJ = DFF // FF

NW = 32
CH = 32


def _router_body(x_ref, wr_ref, i0_ref, i1_ref, g0_ref, g1_ref, ps_ref, ds_ref):
    logits = lax.dot_general(
        x_ref[...].astype(jnp.bfloat16), wr_ref[...].astype(jnp.bfloat16),
        (((1,), (0,)), ((), ())),
        preferred_element_type=jnp.float32,
    )
    m = jnp.max(logits, axis=1, keepdims=True)
    ex = jnp.exp(logits - m)
    probs = ex / jnp.sum(ex, axis=1, keepdims=True)
    iota = lax.broadcasted_iota(jnp.int32, (TT, E), 1)
    v0 = jnp.max(probs, axis=1)
    i0 = jnp.argmax(probs, axis=1).astype(jnp.int32)
    masked = jnp.where(iota == i0[:, None], -1.0, probs)
    v1 = jnp.max(masked, axis=1)
    i1 = jnp.argmax(masked, axis=1).astype(jnp.int32)
    s = v0 + v1
    i0_ref[0, 0, :] = i0
    i1_ref[0, 0, :] = i1
    g0_ref[0, 0, :] = v0 / s
    g1_ref[0, 0, :] = v1 / s
    ps_ref[0, 0, :] = jnp.sum(probs, axis=0)
    oh = (iota == i0[:, None]).astype(jnp.float32) + (iota == i1[:, None]).astype(jnp.float32)
    ds_ref[0, 0, :] = jnp.sum(oh, axis=0)


def _router(x, wr):
    return pl.pallas_call(
        _router_body,
        grid=(NT_R,),
        in_specs=[
            pl.BlockSpec((TT, H), lambda i: (i, 0)),
            pl.BlockSpec((H, E), lambda i: (0, 0)),
        ],
        out_specs=[
            pl.BlockSpec((1, 1, TT), lambda i: (i, 0, 0)),
            pl.BlockSpec((1, 1, TT), lambda i: (i, 0, 0)),
            pl.BlockSpec((1, 1, TT), lambda i: (i, 0, 0)),
            pl.BlockSpec((1, 1, TT), lambda i: (i, 0, 0)),
            pl.BlockSpec((1, 1, E), lambda i: (i, 0, 0)),
            pl.BlockSpec((1, 1, E), lambda i: (i, 0, 0)),
        ],
        out_shape=[
            jax.ShapeDtypeStruct((NT_R, 1, TT), jnp.int32),
            jax.ShapeDtypeStruct((NT_R, 1, TT), jnp.int32),
            jax.ShapeDtypeStruct((NT_R, 1, TT), jnp.float32),
            jax.ShapeDtypeStruct((NT_R, 1, TT), jnp.float32),
            jax.ShapeDtypeStruct((NT_R, 1, E), jnp.float32),
            jax.ShapeDtypeStruct((NT_R, 1, E), jnp.float32),
        ],
    )(x, wr)


def _sc_gather(data, idx):
    n = idx.shape[0]
    d = data.shape[1]
    per_w = n // NW

    mesh = plsc.VectorSubcoreMesh(core_axis_name="c", subcore_axis_name="s")

    @functools.partial(
        pl.kernel,
        mesh=mesh,
        out_type=jax.ShapeDtypeStruct((n, d), data.dtype),
        scratch_types=[
            pltpu.VMEM((CH,), jnp.int32),
            pltpu.VMEM((CH, d), data.dtype),
            pltpu.SemaphoreType.DMA,
        ],
    )
    def gather_kernel(data_hbm, idx_hbm, out_hbm, idx_v, rows_v, sem):
        wid = lax.axis_index("s") * 2 + lax.axis_index("c")
        base = pl.multiple_of(wid * per_w, CH)

        @pl.loop(0, per_w, step=CH)
        def _(off):
            start = pl.multiple_of(base + off, CH)
            pltpu.sync_copy(idx_hbm.at[pl.ds(start, CH)], idx_v)
            pltpu.async_copy(data_hbm.at[idx_v], rows_v, sem).wait()
            pltpu.sync_copy(rows_v, out_hbm.at[pl.ds(start, CH)])

    return gather_kernel(data, idx)


def _mlp_body(te_ref, xg_ref, w1_ref, b1_ref, w2_ref, b2_ref, g_ref, out_ref):
    j = pl.program_id(1)
    xb = xg_ref[...].astype(jnp.bfloat16)
    w1 = w1_ref[0].astype(jnp.bfloat16)
    h = jnp.dot(xb, w1, preferred_element_type=jnp.float32) + b1_ref[0]
    h = jax.nn.gelu(h)
    w2 = w2_ref[0].astype(jnp.bfloat16)
    part = jnp.dot(h.astype(jnp.bfloat16), w2, preferred_element_type=jnp.float32)

    @pl.when(j == 0)
    def _():
        out_ref[...] = jnp.zeros_like(out_ref)

    out_ref[...] += part

    @pl.when(j == NJ - 1)
    def _():
        out_ref[...] = (out_ref[...] + b2_ref[0]) * g_ref[0, 0][:, None]


def _grouped_mlp(tile_expert, xg, w1, b1r, w2, b2r, gater):
    grid_spec = pltpu.PrefetchScalarGridSpec(
        num_scalar_prefetch=1,
        grid=(NT, NJ),
        in_specs=[
            pl.BlockSpec((TM, H), lambda i, j, te: (i, 0)),
            pl.BlockSpec((1, H, FF), lambda i, j, te: (te[i], 0, j)),
            pl.BlockSpec((1, 1, FF), lambda i, j, te: (te[i], 0, j)),
            pl.BlockSpec((1, FF, H), lambda i, j, te: (te[i], j, 0)),
            pl.BlockSpec((1, 1, H), lambda i, j, te: (te[i], 0, 0)),
            pl.BlockSpec((1, 1, TM), lambda i, j, te: (i, 0, 0)),
        ],
        out_specs=pl.BlockSpec((TM, H), lambda i, j, te: (i, 0)),
    )
    return pl.pallas_call(
        _mlp_body,
        grid_spec=grid_spec,
        out_shape=jax.ShapeDtypeStruct((P, H), jnp.float32),
    )(tile_expert, xg, w1, b1r, w2, b2r, gater)


def _add_body(a_ref, b_ref, o_ref):
    o_ref[...] = a_ref[...] + b_ref[...]


def _combine_add(comb):
    return pl.pallas_call(
        _add_body,
        grid=(NT_R,),
        in_specs=[
            pl.BlockSpec((TT, H), lambda i: (i, 0)),
            pl.BlockSpec((TT, H), lambda i: (i + NT_R, 0)),
        ],
        out_specs=pl.BlockSpec((TT, H), lambda i: (i, 0)),
        out_shape=jax.ShapeDtypeStruct((T, H), jnp.float32),
    )(comb, comb)


def kernel(hidden_states, Wr, W1, b1, W2, b2):
    x = hidden_states.reshape(T, H)

    i0_3, i1_3, g0_3, g1_3, ps_3, ds_3 = _router(x, Wr)
    i0 = i0_3.reshape(T)
    i1 = i1_3.reshape(T)
    g0 = g0_3.reshape(T)
    g1 = g1_3.reshape(T)

    e_flat = jnp.stack([i0, i1], axis=1).reshape(-1)
    gate_flat = jnp.stack([g0, g1], axis=1).reshape(-1)
    token_flat = (jnp.arange(T * K, dtype=jnp.int32) // K).astype(jnp.int32)
    oh = (e_flat[:, None] == jnp.arange(E, dtype=jnp.int32)[None, :]).astype(jnp.int32)
    counts = jnp.sum(oh, axis=0)
    rank = jnp.take_along_axis(jnp.cumsum(oh, axis=0) - oh, e_flat[:, None], axis=1)[:, 0]
    padded_counts = ((counts + TM - 1) // TM) * TM
    bounds = jnp.cumsum(padded_counts)
    pstart = bounds - padded_counts
    pos = (pstart[e_flat] + rank).astype(jnp.int32)
    row_token = jnp.zeros((P,), jnp.int32).at[pos].set(token_flat)
    row_gate = jnp.zeros((P,), jnp.float32).at[pos].set(gate_flat)
    tile_expert = jnp.minimum(
        jnp.searchsorted(bounds, jnp.arange(NT, dtype=jnp.int32) * TM, side="right"),
        E - 1,
    ).astype(jnp.int32)
    pos_all = jnp.concatenate([pos[0::2], pos[1::2]])

    psum = jnp.sum(ps_3, axis=(0, 1))
    dsum = jnp.sum(ds_3, axis=(0, 1))
    aux = jnp.float32(E) * jnp.sum((dsum / T) * (psum / T))

    xg = _sc_gather(x, row_token)
    yg = _grouped_mlp(tile_expert, xg, W1, b1.reshape(E, 1, DFF),
                      W2, b2.reshape(E, 1, H), row_gate.reshape(NT, 1, TM))
    comb = _sc_gather(yg, pos_all)
    out = _combine_add(comb)

    return out.reshape(1, T, H), aux


# speedup vs baseline: 1.6118x; 1.2256x over previous
import functools

import jax
import jax.numpy as jnp
from jax import lax
from jax.experimental import pallas as pl
from jax.experimental.pallas import tpu as pltpu
from jax.experimental.pallas import tpu_sc as plsc

T = 2048
H = 2048
E = 8
K = 2
DFF = 8192

TT = 256
NT_R = T // TT

TM = 512
P = 4096 + E * TM // 1
NT = P // TM
FF = 1024
NJ = DFF // FF

NW = 32
CH = 32


def _router_body(x_ref, wr_ref, i0_ref, i1_ref, g0_ref, g1_ref, ps_ref, ds_ref):
    logits = lax.dot_general(
        x_ref[...].astype(jnp.bfloat16), wr_ref[...].astype(jnp.bfloat16),
        (((1,), (0,)), ((), ())),
        preferred_element_type=jnp.float32,
    )
    m = jnp.max(logits, axis=1, keepdims=True)
    ex = jnp.exp(logits - m)
    probs = ex / jnp.sum(ex, axis=1, keepdims=True)
    iota = lax.broadcasted_iota(jnp.int32, (TT, E), 1)
    v0 = jnp.max(probs, axis=1)
    i0 = jnp.argmax(probs, axis=1).astype(jnp.int32)
    masked = jnp.where(iota == i0[:, None], -1.0, probs)
    v1 = jnp.max(masked, axis=1)
    i1 = jnp.argmax(masked, axis=1).astype(jnp.int32)
    s = v0 + v1
    i0_ref[0, 0, :] = i0
    i1_ref[0, 0, :] = i1
    g0_ref[0, 0, :] = v0 / s
    g1_ref[0, 0, :] = v1 / s
    ps_ref[0, 0, :] = jnp.sum(probs, axis=0)
    oh = (iota == i0[:, None]).astype(jnp.float32) + (iota == i1[:, None]).astype(jnp.float32)
    ds_ref[0, 0, :] = jnp.sum(oh, axis=0)


def _router(x, wr):
    return pl.pallas_call(
        _router_body,
        grid=(NT_R,),
        in_specs=[
            pl.BlockSpec((TT, H), lambda i: (i, 0)),
            pl.BlockSpec((H, E), lambda i: (0, 0)),
        ],
        out_specs=[
            pl.BlockSpec((1, 1, TT), lambda i: (i, 0, 0)),
            pl.BlockSpec((1, 1, TT), lambda i: (i, 0, 0)),
            pl.BlockSpec((1, 1, TT), lambda i: (i, 0, 0)),
            pl.BlockSpec((1, 1, TT), lambda i: (i, 0, 0)),
            pl.BlockSpec((1, 1, E), lambda i: (i, 0, 0)),
            pl.BlockSpec((1, 1, E), lambda i: (i, 0, 0)),
        ],
        out_shape=[
            jax.ShapeDtypeStruct((NT_R, 1, TT), jnp.int32),
            jax.ShapeDtypeStruct((NT_R, 1, TT), jnp.int32),
            jax.ShapeDtypeStruct((NT_R, 1, TT), jnp.float32),
            jax.ShapeDtypeStruct((NT_R, 1, TT), jnp.float32),
            jax.ShapeDtypeStruct((NT_R, 1, E), jnp.float32),
            jax.ShapeDtypeStruct((NT_R, 1, E), jnp.float32),
        ],
    )(x, wr)


def _sc_gather(data, idx):
    n = idx.shape[0]
    d = data.shape[1]
    per_w = n // NW

    mesh = plsc.VectorSubcoreMesh(core_axis_name="c", subcore_axis_name="s")

    @functools.partial(
        pl.kernel,
        mesh=mesh,
        out_type=jax.ShapeDtypeStruct((n, d), data.dtype),
        scratch_types=[
            pltpu.VMEM((CH,), jnp.int32),
            pltpu.VMEM((CH, d), data.dtype),
            pltpu.SemaphoreType.DMA,
        ],
    )
    def gather_kernel(data_hbm, idx_hbm, out_hbm, idx_v, rows_v, sem):
        wid = lax.axis_index("s") * 2 + lax.axis_index("c")
        base = pl.multiple_of(wid * per_w, CH)

        @pl.loop(0, per_w, step=CH)
        def _(off):
            start = pl.multiple_of(base + off, CH)
            pltpu.sync_copy(idx_hbm.at[pl.ds(start, CH)], idx_v)
            pltpu.async_copy(data_hbm.at[idx_v], rows_v, sem).wait()
            pltpu.sync_copy(rows_v, out_hbm.at[pl.ds(start, CH)])

    return gather_kernel(data, idx)


def _mlp_body(te_ref, xg_ref, w1_ref, b1_ref, w2_ref, b2_ref, g_ref, out_ref):
    j = pl.program_id(1)
    xb = xg_ref[...].astype(jnp.bfloat16)
    w1 = w1_ref[0].astype(jnp.bfloat16)
    h = jnp.dot(xb, w1, preferred_element_type=jnp.float32) + b1_ref[0]
    h = jax.nn.gelu(h)
    w2 = w2_ref[0].astype(jnp.bfloat16)
    part = jnp.dot(h.astype(jnp.bfloat16), w2, preferred_element_type=jnp.float32)

    @pl.when(j == 0)
    def _():
        out_ref[...] = jnp.zeros_like(out_ref)

    out_ref[...] += part

    @pl.when(j == NJ - 1)
    def _():
        out_ref[...] = (out_ref[...] + b2_ref[0]) * g_ref[0, 0][:, None]


def _grouped_mlp(tile_expert, xg, w1, b1r, w2, b2r, gater):
    grid_spec = pltpu.PrefetchScalarGridSpec(
        num_scalar_prefetch=1,
        grid=(NT, NJ),
        in_specs=[
            pl.BlockSpec((TM, H), lambda i, j, te: (i, 0)),
            pl.BlockSpec((1, H, FF), lambda i, j, te: (te[i], 0, j)),
            pl.BlockSpec((1, 1, FF), lambda i, j, te: (te[i], 0, j)),
            pl.BlockSpec((1, FF, H), lambda i, j, te: (te[i], j, 0)),
            pl.BlockSpec((1, 1, H), lambda i, j, te: (te[i], 0, 0)),
            pl.BlockSpec((1, 1, TM), lambda i, j, te: (i, 0, 0)),
        ],
        out_specs=pl.BlockSpec((TM, H), lambda i, j, te: (i, 0)),
    )
    return pl.pallas_call(
        _mlp_body,
        grid_spec=grid_spec,
        out_shape=jax.ShapeDtypeStruct((P, H), jnp.float32),
    )(tile_expert, xg, w1, b1r, w2, b2r, gater)


def _add_body(a_ref, b_ref, o_ref):
    o_ref[...] = a_ref[...] + b_ref[...]


def _combine_add(comb):
    return pl.pallas_call(
        _add_body,
        grid=(NT_R,),
        in_specs=[
            pl.BlockSpec((TT, H), lambda i: (i, 0)),
            pl.BlockSpec((TT, H), lambda i: (i + NT_R, 0)),
        ],
        out_specs=pl.BlockSpec((TT, H), lambda i: (i, 0)),
        out_shape=jax.ShapeDtypeStruct((T, H), jnp.float32),
    )(comb, comb)


def kernel(hidden_states, Wr, W1, b1, W2, b2):
    x = hidden_states.reshape(T, H)

    i0_3, i1_3, g0_3, g1_3, ps_3, ds_3 = _router(x, Wr)
    i0 = i0_3.reshape(T)
    i1 = i1_3.reshape(T)
    g0 = g0_3.reshape(T)
    g1 = g1_3.reshape(T)

    e_flat = jnp.stack([i0, i1], axis=1).reshape(-1)
    gate_flat = jnp.stack([g0, g1], axis=1).reshape(-1)
    token_flat = (jnp.arange(T * K, dtype=jnp.int32) // K).astype(jnp.int32)
    oh = (e_flat[:, None] == jnp.arange(E, dtype=jnp.int32)[None, :]).astype(jnp.int32)
    counts = jnp.sum(oh, axis=0)
    rank = jnp.take_along_axis(jnp.cumsum(oh, axis=0) - oh, e_flat[:, None], axis=1)[:, 0]
    padded_counts = ((counts + TM - 1) // TM) * TM
    bounds = jnp.cumsum(padded_counts)
    pstart = bounds - padded_counts
    pos = (pstart[e_flat] + rank).astype(jnp.int32)
    filler = (jnp.arange(P, dtype=jnp.int32) % T).astype(jnp.int32)
    row_token = filler.at[pos].set(token_flat)
    row_gate = jnp.zeros((P,), jnp.float32).at[pos].set(gate_flat)
    tile_expert = jnp.minimum(
        jnp.searchsorted(bounds, jnp.arange(NT, dtype=jnp.int32) * TM, side="right"),
        E - 1,
    ).astype(jnp.int32)
    pos_all = jnp.concatenate([pos[0::2], pos[1::2]])

    psum = jnp.sum(ps_3, axis=(0, 1))
    dsum = jnp.sum(ds_3, axis=(0, 1))
    aux = jnp.float32(E) * jnp.sum((dsum / T) * (psum / T))

    xg = _sc_gather(x, row_token)
    yg = _grouped_mlp(tile_expert, xg, W1, b1.reshape(E, 1, DFF),
                      W2, b2.reshape(E, 1, H), row_gate.reshape(NT, 1, TM))
    comb = _sc_gather(yg, pos_all)
    out = _combine_add(comb)

    return out.reshape(1, T, H), aux


# speedup vs baseline: 1.6556x; 1.0272x over previous
import functools

import jax
import jax.numpy as jnp
from jax import lax
from jax.experimental import pallas as pl
from jax.experimental.pallas import tpu as pltpu
from jax.experimental.pallas import tpu_sc as plsc

T = 2048
H = 2048
E = 8
K = 2
DFF = 8192

TT = 256
NT_R = T // TT

TM = 512
P = 4096 + E * TM // 1
NT = P // TM
FF = 1024
NJ = DFF // FF

NW = 32
CH = 32


def _router_body(x_ref, wr_ref, i0_ref, i1_ref, g0_ref, g1_ref, ps_ref, ds_ref):
    logits = lax.dot_general(
        x_ref[...].astype(jnp.bfloat16), wr_ref[...].astype(jnp.bfloat16),
        (((1,), (0,)), ((), ())),
        preferred_element_type=jnp.float32,
    )
    m = jnp.max(logits, axis=1, keepdims=True)
    ex = jnp.exp(logits - m)
    probs = ex / jnp.sum(ex, axis=1, keepdims=True)
    iota = lax.broadcasted_iota(jnp.int32, (TT, E), 1)
    v0 = jnp.max(probs, axis=1)
    i0 = jnp.argmax(probs, axis=1).astype(jnp.int32)
    masked = jnp.where(iota == i0[:, None], -1.0, probs)
    v1 = jnp.max(masked, axis=1)
    i1 = jnp.argmax(masked, axis=1).astype(jnp.int32)
    s = v0 + v1
    i0_ref[0, 0, :] = i0
    i1_ref[0, 0, :] = i1
    g0_ref[0, 0, :] = v0 / s
    g1_ref[0, 0, :] = v1 / s
    ps_ref[0, 0, :] = jnp.sum(probs, axis=0)
    oh = (iota == i0[:, None]).astype(jnp.float32) + (iota == i1[:, None]).astype(jnp.float32)
    ds_ref[0, 0, :] = jnp.sum(oh, axis=0)


def _router(x, wr):
    return pl.pallas_call(
        _router_body,
        grid=(NT_R,),
        in_specs=[
            pl.BlockSpec((TT, H), lambda i: (i, 0)),
            pl.BlockSpec((H, E), lambda i: (0, 0)),
        ],
        out_specs=[
            pl.BlockSpec((1, 1, TT), lambda i: (i, 0, 0)),
            pl.BlockSpec((1, 1, TT), lambda i: (i, 0, 0)),
            pl.BlockSpec((1, 1, TT), lambda i: (i, 0, 0)),
            pl.BlockSpec((1, 1, TT), lambda i: (i, 0, 0)),
            pl.BlockSpec((1, 1, E), lambda i: (i, 0, 0)),
            pl.BlockSpec((1, 1, E), lambda i: (i, 0, 0)),
        ],
        out_shape=[
            jax.ShapeDtypeStruct((NT_R, 1, TT), jnp.int32),
            jax.ShapeDtypeStruct((NT_R, 1, TT), jnp.int32),
            jax.ShapeDtypeStruct((NT_R, 1, TT), jnp.float32),
            jax.ShapeDtypeStruct((NT_R, 1, TT), jnp.float32),
            jax.ShapeDtypeStruct((NT_R, 1, E), jnp.float32),
            jax.ShapeDtypeStruct((NT_R, 1, E), jnp.float32),
        ],
    )(x, wr)


def _sc_gather(data, idx):
    n = idx.shape[0]
    d = data.shape[1]
    per_w = n // NW

    mesh = plsc.VectorSubcoreMesh(core_axis_name="c", subcore_axis_name="s")

    @functools.partial(
        pl.kernel,
        mesh=mesh,
        out_type=jax.ShapeDtypeStruct((n, d), data.dtype),
        scratch_types=[
            pltpu.VMEM((CH,), jnp.int32),
            pltpu.VMEM((CH, d), data.dtype),
            pltpu.SemaphoreType.DMA,
        ],
    )
    def gather_kernel(data_hbm, idx_hbm, out_hbm, idx_v, rows_v, sem):
        wid = lax.axis_index("s") * 2 + lax.axis_index("c")
        base = pl.multiple_of(wid * per_w, CH)

        @pl.loop(0, per_w, step=CH)
        def _(off):
            start = pl.multiple_of(base + off, CH)
            pltpu.sync_copy(idx_hbm.at[pl.ds(start, CH)], idx_v)
            pltpu.async_copy(data_hbm.at[idx_v], rows_v, sem).wait()
            pltpu.sync_copy(rows_v, out_hbm.at[pl.ds(start, CH)])

    return gather_kernel(data, idx)


def _mlp_body(te_ref, xg_ref, w1_ref, b1_ref, w2_ref, b2_ref, g_ref, out_ref):
    i = pl.program_id(0)
    j = pl.program_id(1)

    @pl.when(i < te_ref[NT])
    def _():
        xb = xg_ref[...].astype(jnp.bfloat16)
        w1 = w1_ref[0].astype(jnp.bfloat16)
        h = jnp.dot(xb, w1, preferred_element_type=jnp.float32) + b1_ref[0]
        h = jax.nn.gelu(h)
        w2 = w2_ref[0].astype(jnp.bfloat16)
        part = jnp.dot(h.astype(jnp.bfloat16), w2, preferred_element_type=jnp.float32)

        @pl.when(j == 0)
        def _():
            out_ref[...] = jnp.zeros_like(out_ref)

        out_ref[...] += part

        @pl.when(j == NJ - 1)
        def _():
            out_ref[...] = (out_ref[...] + b2_ref[0]) * g_ref[0, 0][:, None]


def _grouped_mlp(tile_expert, xg, w1, b1r, w2, b2r, gater):
    grid_spec = pltpu.PrefetchScalarGridSpec(
        num_scalar_prefetch=1,
        grid=(NT, NJ),
        in_specs=[
            pl.BlockSpec((TM, H), lambda i, j, te: (i, 0)),
            pl.BlockSpec((1, H, FF), lambda i, j, te: (te[i], 0, j)),
            pl.BlockSpec((1, 1, FF), lambda i, j, te: (te[i], 0, j)),
            pl.BlockSpec((1, FF, H), lambda i, j, te: (te[i], j, 0)),
            pl.BlockSpec((1, 1, H), lambda i, j, te: (te[i], 0, 0)),
            pl.BlockSpec((1, 1, TM), lambda i, j, te: (i, 0, 0)),
        ],
        out_specs=pl.BlockSpec((TM, H), lambda i, j, te: (i, 0)),
    )
    return pl.pallas_call(
        _mlp_body,
        grid_spec=grid_spec,
        out_shape=jax.ShapeDtypeStruct((P, H), jnp.float32),
    )(tile_expert, xg, w1, b1r, w2, b2r, gater)


def _add_body(a_ref, b_ref, o_ref):
    o_ref[...] = a_ref[...] + b_ref[...]


def _combine_add(comb):
    return pl.pallas_call(
        _add_body,
        grid=(NT_R,),
        in_specs=[
            pl.BlockSpec((TT, H), lambda i: (i, 0)),
            pl.BlockSpec((TT, H), lambda i: (i + NT_R, 0)),
        ],
        out_specs=pl.BlockSpec((TT, H), lambda i: (i, 0)),
        out_shape=jax.ShapeDtypeStruct((T, H), jnp.float32),
    )(comb, comb)


def kernel(hidden_states, Wr, W1, b1, W2, b2):
    x = hidden_states.reshape(T, H)

    i0_3, i1_3, g0_3, g1_3, ps_3, ds_3 = _router(x, Wr)
    i0 = i0_3.reshape(T)
    i1 = i1_3.reshape(T)
    g0 = g0_3.reshape(T)
    g1 = g1_3.reshape(T)

    e_flat = jnp.stack([i0, i1], axis=1).reshape(-1)
    gate_flat = jnp.stack([g0, g1], axis=1).reshape(-1)
    token_flat = (jnp.arange(T * K, dtype=jnp.int32) // K).astype(jnp.int32)
    oh = (e_flat[:, None] == jnp.arange(E, dtype=jnp.int32)[None, :]).astype(jnp.int32)
    counts = jnp.sum(oh, axis=0)
    rank = jnp.take_along_axis(jnp.cumsum(oh, axis=0) - oh, e_flat[:, None], axis=1)[:, 0]
    padded_counts = ((counts + TM - 1) // TM) * TM
    bounds = jnp.cumsum(padded_counts)
    pstart = bounds - padded_counts
    pos = (pstart[e_flat] + rank).astype(jnp.int32)
    filler = (jnp.arange(P, dtype=jnp.int32) % T).astype(jnp.int32)
    row_token = filler.at[pos].set(token_flat)
    row_gate = jnp.zeros((P,), jnp.float32).at[pos].set(gate_flat)
    tile_expert = jnp.minimum(
        jnp.searchsorted(bounds, jnp.arange(NT, dtype=jnp.int32) * TM, side="right"),
        E - 1,
    ).astype(jnp.int32)
    n_used_tiles = (bounds[E - 1] + TM - 1) // TM
    tile_meta = jnp.concatenate([tile_expert, n_used_tiles[None].astype(jnp.int32)])
    pos_all = jnp.concatenate([pos[0::2], pos[1::2]])

    psum = jnp.sum(ps_3, axis=(0, 1))
    dsum = jnp.sum(ds_3, axis=(0, 1))
    aux = jnp.float32(E) * jnp.sum((dsum / T) * (psum / T))

    xg = _sc_gather(x, row_token)
    yg = _grouped_mlp(tile_meta, xg, W1, b1.reshape(E, 1, DFF),
                      W2, b2.reshape(E, 1, H), row_gate.reshape(NT, 1, TM))
    comb = _sc_gather(yg, pos_all)
    out = _combine_add(comb)

    return out.reshape(1, T, H), aux


# speedup vs baseline: 1.8784x; 1.1346x over previous
import functools

import jax
import jax.numpy as jnp
from jax import lax
from jax.experimental import pallas as pl
from jax.experimental.pallas import tpu as pltpu
from jax.experimental.pallas import tpu_sc as plsc

T = 2048
H = 2048
E = 8
K = 2
DFF = 8192

TT = 256
NT_R = T // TT

TM = 512
P = 4096 + E * TM // 1
NT = P // TM
FF = 1024
NJ = DFF // FF

NW = 32
CH = 32


def _router_body(x_ref, wr_ref, i0_ref, i1_ref, g0_ref, g1_ref, ps_ref, ds_ref):
    logits = lax.dot_general(
        x_ref[...].astype(jnp.bfloat16), wr_ref[...].astype(jnp.bfloat16),
        (((1,), (0,)), ((), ())),
        preferred_element_type=jnp.float32,
    )
    m = jnp.max(logits, axis=1, keepdims=True)
    ex = jnp.exp(logits - m)
    probs = ex / jnp.sum(ex, axis=1, keepdims=True)
    iota = lax.broadcasted_iota(jnp.int32, (TT, E), 1)
    v0 = jnp.max(probs, axis=1)
    i0 = jnp.argmax(probs, axis=1).astype(jnp.int32)
    masked = jnp.where(iota == i0[:, None], -1.0, probs)
    v1 = jnp.max(masked, axis=1)
    i1 = jnp.argmax(masked, axis=1).astype(jnp.int32)
    s = v0 + v1
    i0_ref[0, 0, :] = i0
    i1_ref[0, 0, :] = i1
    g0_ref[0, 0, :] = v0 / s
    g1_ref[0, 0, :] = v1 / s
    ps_ref[0, 0, :] = jnp.sum(probs, axis=0)
    oh = (iota == i0[:, None]).astype(jnp.float32) + (iota == i1[:, None]).astype(jnp.float32)
    ds_ref[0, 0, :] = jnp.sum(oh, axis=0)


def _router(x, wr):
    return pl.pallas_call(
        _router_body,
        grid=(NT_R,),
        in_specs=[
            pl.BlockSpec((TT, H), lambda i: (i, 0)),
            pl.BlockSpec((H, E), lambda i: (0, 0)),
        ],
        out_specs=[
            pl.BlockSpec((1, 1, TT), lambda i: (i, 0, 0)),
            pl.BlockSpec((1, 1, TT), lambda i: (i, 0, 0)),
            pl.BlockSpec((1, 1, TT), lambda i: (i, 0, 0)),
            pl.BlockSpec((1, 1, TT), lambda i: (i, 0, 0)),
            pl.BlockSpec((1, 1, E), lambda i: (i, 0, 0)),
            pl.BlockSpec((1, 1, E), lambda i: (i, 0, 0)),
        ],
        out_shape=[
            jax.ShapeDtypeStruct((NT_R, 1, TT), jnp.int32),
            jax.ShapeDtypeStruct((NT_R, 1, TT), jnp.int32),
            jax.ShapeDtypeStruct((NT_R, 1, TT), jnp.float32),
            jax.ShapeDtypeStruct((NT_R, 1, TT), jnp.float32),
            jax.ShapeDtypeStruct((NT_R, 1, E), jnp.float32),
            jax.ShapeDtypeStruct((NT_R, 1, E), jnp.float32),
        ],
    )(x, wr)


def _sc_gather(data, idx):
    n = idx.shape[0]
    d = data.shape[1]
    per_w = n // NW

    mesh = plsc.VectorSubcoreMesh(core_axis_name="c", subcore_axis_name="s")

    @functools.partial(
        pl.kernel,
        mesh=mesh,
        out_type=jax.ShapeDtypeStruct((n, d), data.dtype),
        scratch_types=[
            pltpu.VMEM((CH,), jnp.int32),
            pltpu.VMEM((CH, d), data.dtype),
            pltpu.SemaphoreType.DMA,
        ],
    )
    def gather_kernel(data_hbm, idx_hbm, out_hbm, idx_v, rows_v, sem):
        wid = lax.axis_index("s") * 2 + lax.axis_index("c")
        base = pl.multiple_of(wid * per_w, CH)

        @pl.loop(0, per_w, step=CH)
        def _(off):
            start = pl.multiple_of(base + off, CH)
            pltpu.sync_copy(idx_hbm.at[pl.ds(start, CH)], idx_v)
            pltpu.async_copy(data_hbm.at[idx_v], rows_v, sem).wait()
            pltpu.sync_copy(rows_v, out_hbm.at[pl.ds(start, CH)])

    return gather_kernel(data, idx)


def _mlp_body(te_ref, xg_ref, w1_ref, b1_ref, w2_ref, b2_ref, g_ref, out_ref):
    i = pl.program_id(0)
    j = pl.program_id(1)

    @pl.when(i < te_ref[NT])
    def _():
        xb = xg_ref[...].astype(jnp.bfloat16)
        w1 = w1_ref[0].astype(jnp.bfloat16)
        h = jnp.dot(xb, w1, preferred_element_type=jnp.float32) + b1_ref[0]
        h = jax.nn.gelu(h)
        w2 = w2_ref[0].astype(jnp.bfloat16)
        part = jnp.dot(h.astype(jnp.bfloat16), w2, preferred_element_type=jnp.float32)

        @pl.when(j == 0)
        def _():
            out_ref[...] = jnp.zeros_like(out_ref)

        out_ref[...] += part

        @pl.when(j == NJ - 1)
        def _():
            out_ref[...] = (out_ref[...] + b2_ref[0]) * g_ref[0, 0][:, None]


def _grouped_mlp(tile_expert, xg, w1, b1r, w2, b2r, gater):
    grid_spec = pltpu.PrefetchScalarGridSpec(
        num_scalar_prefetch=1,
        grid=(NT, NJ),
        in_specs=[
            pl.BlockSpec((TM, H), lambda i, j, te: (i, 0)),
            pl.BlockSpec((1, H, FF),
                         lambda i, j, te: (te[i], 0, jnp.where(i < te[NT], j, 0))),
            pl.BlockSpec((1, 1, FF),
                         lambda i, j, te: (te[i], 0, jnp.where(i < te[NT], j, 0))),
            pl.BlockSpec((1, FF, H),
                         lambda i, j, te: (te[i], jnp.where(i < te[NT], j, 0), 0)),
            pl.BlockSpec((1, 1, H), lambda i, j, te: (te[i], 0, 0)),
            pl.BlockSpec((1, 1, TM), lambda i, j, te: (i, 0, 0)),
        ],
        out_specs=pl.BlockSpec((TM, H), lambda i, j, te: (i, 0)),
    )
    return pl.pallas_call(
        _mlp_body,
        grid_spec=grid_spec,
        out_shape=jax.ShapeDtypeStruct((P, H), jnp.float32),
    )(tile_expert, xg, w1, b1r, w2, b2r, gater)


def _add_body(a_ref, b_ref, o_ref):
    o_ref[...] = a_ref[...] + b_ref[...]


def _combine_add(comb):
    return pl.pallas_call(
        _add_body,
        grid=(NT_R,),
        in_specs=[
            pl.BlockSpec((TT, H), lambda i: (i, 0)),
            pl.BlockSpec((TT, H), lambda i: (i + NT_R, 0)),
        ],
        out_specs=pl.BlockSpec((TT, H), lambda i: (i, 0)),
        out_shape=jax.ShapeDtypeStruct((T, H), jnp.float32),
    )(comb, comb)


def kernel(hidden_states, Wr, W1, b1, W2, b2):
    x = hidden_states.reshape(T, H)

    i0_3, i1_3, g0_3, g1_3, ps_3, ds_3 = _router(x, Wr)
    i0 = i0_3.reshape(T)
    i1 = i1_3.reshape(T)
    g0 = g0_3.reshape(T)
    g1 = g1_3.reshape(T)

    e_flat = jnp.stack([i0, i1], axis=1).reshape(-1)
    gate_flat = jnp.stack([g0, g1], axis=1).reshape(-1)
    token_flat = (jnp.arange(T * K, dtype=jnp.int32) // K).astype(jnp.int32)
    oh = (e_flat[:, None] == jnp.arange(E, dtype=jnp.int32)[None, :]).astype(jnp.int32)
    counts = jnp.sum(oh, axis=0)
    rank = jnp.take_along_axis(jnp.cumsum(oh, axis=0) - oh, e_flat[:, None], axis=1)[:, 0]
    padded_counts = ((counts + TM - 1) // TM) * TM
    bounds = jnp.cumsum(padded_counts)
    pstart = bounds - padded_counts
    pos = (pstart[e_flat] + rank).astype(jnp.int32)
    filler = (jnp.arange(P, dtype=jnp.int32) % T).astype(jnp.int32)
    row_token = filler.at[pos].set(token_flat)
    row_gate = jnp.zeros((P,), jnp.float32).at[pos].set(gate_flat)
    tile_expert = jnp.minimum(
        jnp.searchsorted(bounds, jnp.arange(NT, dtype=jnp.int32) * TM, side="right"),
        E - 1,
    ).astype(jnp.int32)
    n_used_tiles = (bounds[E - 1] + TM - 1) // TM
    tile_meta = jnp.concatenate([tile_expert, n_used_tiles[None].astype(jnp.int32)])
    pos_all = jnp.concatenate([pos[0::2], pos[1::2]])

    psum = jnp.sum(ps_3, axis=(0, 1))
    dsum = jnp.sum(ds_3, axis=(0, 1))
    aux = jnp.float32(E) * jnp.sum((dsum / T) * (psum / T))

    xg = _sc_gather(x, row_token)
    yg = _grouped_mlp(tile_meta, xg, W1, b1.reshape(E, 1, DFF),
                      W2, b2.reshape(E, 1, H), row_gate.reshape(NT, 1, TM))
    comb = _sc_gather(yg, pos_all)
    out = _combine_add(comb)

    return out.reshape(1, T, H), aux


# speedup vs baseline: 1.9193x; 1.0218x over previous
import functools

import jax
import jax.numpy as jnp
from jax import lax
from jax.experimental import pallas as pl
from jax.experimental.pallas import tpu as pltpu
from jax.experimental.pallas import tpu_sc as plsc

T = 2048
H = 2048
E = 8
K = 2
DFF = 8192

TT = 256
NT_R = T // TT

TM = 512
P = 4096 + E * TM // 1
NT = P // TM
FF = 1024
NJ = DFF // FF

NW = 32
CH = 32


def _router_body(x_ref, wr_ref, i0_ref, i1_ref, g0_ref, g1_ref, ps_ref, ds_ref):
    logits = lax.dot_general(
        x_ref[...].astype(jnp.bfloat16), wr_ref[...].astype(jnp.bfloat16),
        (((1,), (0,)), ((), ())),
        preferred_element_type=jnp.float32,
    )
    m = jnp.max(logits, axis=1, keepdims=True)
    ex = jnp.exp(logits - m)
    probs = ex / jnp.sum(ex, axis=1, keepdims=True)
    iota = lax.broadcasted_iota(jnp.int32, (TT, E), 1)
    v0 = jnp.max(probs, axis=1)
    i0 = jnp.argmax(probs, axis=1).astype(jnp.int32)
    masked = jnp.where(iota == i0[:, None], -1.0, probs)
    v1 = jnp.max(masked, axis=1)
    i1 = jnp.argmax(masked, axis=1).astype(jnp.int32)
    s = v0 + v1
    i0_ref[0, 0, :] = i0
    i1_ref[0, 0, :] = i1
    g0_ref[0, 0, :] = v0 / s
    g1_ref[0, 0, :] = v1 / s
    ps_ref[0, 0, :] = jnp.sum(probs, axis=0)
    oh = (iota == i0[:, None]).astype(jnp.float32) + (iota == i1[:, None]).astype(jnp.float32)
    ds_ref[0, 0, :] = jnp.sum(oh, axis=0)


def _router(x, wr):
    return pl.pallas_call(
        _router_body,
        grid=(NT_R,),
        in_specs=[
            pl.BlockSpec((TT, H), lambda i: (i, 0)),
            pl.BlockSpec((H, E), lambda i: (0, 0)),
        ],
        out_specs=[
            pl.BlockSpec((1, 1, TT), lambda i: (i, 0, 0)),
            pl.BlockSpec((1, 1, TT), lambda i: (i, 0, 0)),
            pl.BlockSpec((1, 1, TT), lambda i: (i, 0, 0)),
            pl.BlockSpec((1, 1, TT), lambda i: (i, 0, 0)),
            pl.BlockSpec((1, 1, E), lambda i: (i, 0, 0)),
            pl.BlockSpec((1, 1, E), lambda i: (i, 0, 0)),
        ],
        out_shape=[
            jax.ShapeDtypeStruct((NT_R, 1, TT), jnp.int32),
            jax.ShapeDtypeStruct((NT_R, 1, TT), jnp.int32),
            jax.ShapeDtypeStruct((NT_R, 1, TT), jnp.float32),
            jax.ShapeDtypeStruct((NT_R, 1, TT), jnp.float32),
            jax.ShapeDtypeStruct((NT_R, 1, E), jnp.float32),
            jax.ShapeDtypeStruct((NT_R, 1, E), jnp.float32),
        ],
    )(x, wr)


def _sc_gather(data, idx):
    n = idx.shape[0]
    d = data.shape[1]
    per_w = n // NW

    mesh = plsc.VectorSubcoreMesh(core_axis_name="c", subcore_axis_name="s")

    @functools.partial(
        pl.kernel,
        mesh=mesh,
        out_type=jax.ShapeDtypeStruct((n, d), data.dtype),
        scratch_types=[
            pltpu.VMEM((CH,), jnp.int32),
            pltpu.VMEM((CH, d), data.dtype),
            pltpu.SemaphoreType.DMA,
        ],
    )
    def gather_kernel(data_hbm, idx_hbm, out_hbm, idx_v, rows_v, sem):
        wid = lax.axis_index("s") * 2 + lax.axis_index("c")
        base = pl.multiple_of(wid * per_w, CH)

        @pl.loop(0, per_w, step=CH)
        def _(off):
            start = pl.multiple_of(base + off, CH)
            pltpu.sync_copy(idx_hbm.at[pl.ds(start, CH)], idx_v)
            pltpu.async_copy(data_hbm.at[idx_v], rows_v, sem).wait()
            pltpu.sync_copy(rows_v, out_hbm.at[pl.ds(start, CH)])

    return gather_kernel(data, idx)


def _mlp_body(te_ref, xg_ref, w1_ref, b1_ref, w2_ref, b2_ref, g_ref, out_ref):
    i = pl.program_id(0)
    j = pl.program_id(1)

    @pl.when(i < te_ref[NT])
    def _():
        xb = xg_ref[...]
        w1 = w1_ref[0]
        h = jnp.dot(xb, w1, preferred_element_type=jnp.float32) + b1_ref[0]
        h = jax.nn.gelu(h)
        w2 = w2_ref[0]
        part = jnp.dot(h, w2, preferred_element_type=jnp.float32)

        @pl.when(j == 0)
        def _():
            out_ref[...] = jnp.zeros_like(out_ref)

        out_ref[...] += part

        @pl.when(j == NJ - 1)
        def _():
            out_ref[...] = (out_ref[...] + b2_ref[0]) * g_ref[0, 0][:, None]


def _grouped_mlp(tile_expert, xg, w1, b1r, w2, b2r, gater):
    grid_spec = pltpu.PrefetchScalarGridSpec(
        num_scalar_prefetch=1,
        grid=(NT, NJ),
        in_specs=[
            pl.BlockSpec((TM, H), lambda i, j, te: (i, 0)),
            pl.BlockSpec((1, H, FF),
                         lambda i, j, te: (te[i], 0, jnp.where(i < te[NT], j, 0))),
            pl.BlockSpec((1, 1, FF),
                         lambda i, j, te: (te[i], 0, jnp.where(i < te[NT], j, 0))),
            pl.BlockSpec((1, FF, H),
                         lambda i, j, te: (te[i], jnp.where(i < te[NT], j, 0), 0)),
            pl.BlockSpec((1, 1, H), lambda i, j, te: (te[i], 0, 0)),
            pl.BlockSpec((1, 1, TM), lambda i, j, te: (i, 0, 0)),
        ],
        out_specs=pl.BlockSpec((TM, H), lambda i, j, te: (i, 0)),
    )
    return pl.pallas_call(
        _mlp_body,
        grid_spec=grid_spec,
        out_shape=jax.ShapeDtypeStruct((P, H), jnp.float32),
    )(tile_expert, xg, w1, b1r, w2, b2r, gater)


def _add_body(a_ref, b_ref, o_ref):
    o_ref[...] = a_ref[...] + b_ref[...]


def _combine_add(comb):
    return pl.pallas_call(
        _add_body,
        grid=(NT_R,),
        in_specs=[
            pl.BlockSpec((TT, H), lambda i: (i, 0)),
            pl.BlockSpec((TT, H), lambda i: (i + NT_R, 0)),
        ],
        out_specs=pl.BlockSpec((TT, H), lambda i: (i, 0)),
        out_shape=jax.ShapeDtypeStruct((T, H), jnp.float32),
    )(comb, comb)


def kernel(hidden_states, Wr, W1, b1, W2, b2):
    x = hidden_states.reshape(T, H)

    i0_3, i1_3, g0_3, g1_3, ps_3, ds_3 = _router(x, Wr)
    i0 = i0_3.reshape(T)
    i1 = i1_3.reshape(T)
    g0 = g0_3.reshape(T)
    g1 = g1_3.reshape(T)

    e_flat = jnp.stack([i0, i1], axis=1).reshape(-1)
    gate_flat = jnp.stack([g0, g1], axis=1).reshape(-1)
    token_flat = (jnp.arange(T * K, dtype=jnp.int32) // K).astype(jnp.int32)
    oh = (e_flat[:, None] == jnp.arange(E, dtype=jnp.int32)[None, :]).astype(jnp.int32)
    counts = jnp.sum(oh, axis=0)
    rank = jnp.take_along_axis(jnp.cumsum(oh, axis=0) - oh, e_flat[:, None], axis=1)[:, 0]
    padded_counts = ((counts + TM - 1) // TM) * TM
    bounds = jnp.cumsum(padded_counts)
    pstart = bounds - padded_counts
    pos = (pstart[e_flat] + rank).astype(jnp.int32)
    filler = (jnp.arange(P, dtype=jnp.int32) % T).astype(jnp.int32)
    row_token = filler.at[pos].set(token_flat)
    row_gate = jnp.zeros((P,), jnp.float32).at[pos].set(gate_flat)
    tile_expert = jnp.minimum(
        jnp.searchsorted(bounds, jnp.arange(NT, dtype=jnp.int32) * TM, side="right"),
        E - 1,
    ).astype(jnp.int32)
    n_used_tiles = (bounds[E - 1] + TM - 1) // TM
    tile_meta = jnp.concatenate([tile_expert, n_used_tiles[None].astype(jnp.int32)])
    pos_all = jnp.concatenate([pos[0::2], pos[1::2]])

    psum = jnp.sum(ps_3, axis=(0, 1))
    dsum = jnp.sum(ds_3, axis=(0, 1))
    aux = jnp.float32(E) * jnp.sum((dsum / T) * (psum / T))

    xg = _sc_gather(x, row_token)
    yg = _grouped_mlp(tile_meta, xg, W1, b1.reshape(E, 1, DFF),
                      W2, b2.reshape(E, 1, H), row_gate.reshape(NT, 1, TM))
    comb = _sc_gather(yg, pos_all)
    out = _combine_add(comb)

    return out.reshape(1, T, H), aux
